# Initial kernel scaffold; baseline (speedup 1.0000x reference)
import jax
import jax.numpy as jnp
from jax.experimental import pallas as pl


def kernel(x_main, x_mod, x_proj, csr_idx, Q_w, Q_b, p1_w, p1_b, bn1_g, bn1_b, p2_w, p2_b, bn2_g, bn2_b, K_w, K_b):
    raise NotImplementedError("write your pallas kernel here")


# speedup vs baseline: 4.7118x; 4.7118x over previous
import dataclasses
import functools

import jax
import jax.numpy as jnp
from jax import lax
from jax.experimental import pallas as pl
from jax.experimental.pallas import tpu as pltpu
from jax.experimental.pallas import tpu_sc as plsc

F32 = jnp.float32

N = 10000
V = 320000
BV = 2000

NC = 2
NS = 16
W = NC * NS
SPW = 320
NPAD = W * SPW
CSR_PAD = NPAD + 16
CH = 32


def _k_y1(xp_ref, w_ref, b_ref, y1_ref, st_ref):
    i = pl.program_id(0)
    y = jnp.dot(xp_ref[...], w_ref[...], preferred_element_type=F32) + b_ref[...]
    y1_ref[...] = y

    @pl.when(i == 0)
    def _():
        st_ref[...] = jnp.zeros_like(st_ref)

    s = jnp.sum(y, axis=0, keepdims=True)
    sq = jnp.sum(y * y, axis=0, keepdims=True)
    pad = jnp.zeros((6, s.shape[1]), F32)
    st_ref[...] = st_ref[...] + jnp.concatenate([s, sq, pad], axis=0)


def _bn_affine(st_ref, g_ref, b_ref):
    inv_v = F32(1.0 / V)
    mean = st_ref[0:1, :] * inv_v
    var = st_ref[1:2, :] * inv_v - mean * mean
    a = g_ref[...] * lax.rsqrt(var + 1e-5)
    c = b_ref[...] - mean * a
    return a, c


def _k_y2(y1_ref, st1_ref, g1_ref, bb1_ref, w_ref, b_ref, y2_ref, st_ref):
    i = pl.program_id(0)
    a, c = _bn_affine(st1_ref, g1_ref, bb1_ref)
    h = jnp.maximum(y1_ref[...] * a + c, 0.0)
    y2 = jnp.dot(h, w_ref[...], preferred_element_type=F32) + b_ref[...]
    y2_ref[...] = y2

    @pl.when(i == 0)
    def _():
        st_ref[...] = jnp.zeros_like(st_ref)

    s = jnp.sum(y2, axis=0, keepdims=True)
    sq = jnp.sum(y2 * y2, axis=0, keepdims=True)
    pad = jnp.zeros((6, s.shape[1]), F32)
    st_ref[...] = st_ref[...] + jnp.concatenate([s, sq, pad], axis=0)


def _k_kf(y2_ref, st2_ref, g2_ref, bb2_ref, xm_ref, wh_ref, wm_ref, kb_ref,
          kf_ref):
    a, c = _bn_affine(st2_ref, g2_ref, bb2_ref)
    h2 = jnp.maximum(y2_ref[...] * a + c, 0.0)
    kf = jnp.dot(h2, wh_ref[...], preferred_element_type=F32)
    kf = kf + jnp.dot(xm_ref[...], wm_ref[...], preferred_element_type=F32)
    kf_ref[...] = kf + kb_ref[...]


def _k_q(xmain_ref, w_ref, b_ref, q_ref):
    q_ref[...] = (jnp.dot(xmain_ref[...], w_ref[...],
                          preferred_element_type=F32) + b_ref[...])


def _dense_stage(x_main, x_mod, x_proj, Q_w, Q_b, p1_w, p1_b, bn1_g, bn1_b,
                 p2_w, p2_b, bn2_g, bn2_b, K_w, K_b):
    grid = (V // BV,)
    row = lambda i: (i, 0)
    const = lambda i: (0, 0)

    y1, st1 = pl.pallas_call(
        _k_y1,
        grid=grid,
        in_specs=[
            pl.BlockSpec((BV, 128), row),
            pl.BlockSpec((128, 128), const),
            pl.BlockSpec((1, 128), const),
        ],
        out_specs=[
            pl.BlockSpec((BV, 128), row),
            pl.BlockSpec((8, 128), const),
        ],
        out_shape=[
            jax.ShapeDtypeStruct((V, 128), F32),
            jax.ShapeDtypeStruct((8, 128), F32),
        ],
    )(x_proj, p1_w.T, p1_b.reshape(1, 128))

    y2, st2 = pl.pallas_call(
        _k_y2,
        grid=grid,
        in_specs=[
            pl.BlockSpec((BV, 128), row),
            pl.BlockSpec((8, 128), const),
            pl.BlockSpec((1, 128), const),
            pl.BlockSpec((1, 128), const),
            pl.BlockSpec((128, 64), const),
            pl.BlockSpec((1, 64), const),
        ],
        out_specs=[
            pl.BlockSpec((BV, 64), row),
            pl.BlockSpec((8, 64), const),
        ],
        out_shape=[
            jax.ShapeDtypeStruct((V, 64), F32),
            jax.ShapeDtypeStruct((8, 64), F32),
        ],
    )(y1, st1, bn1_g.reshape(1, 128), bn1_b.reshape(1, 128), p2_w.T,
      p2_b.reshape(1, 64))

    kf = pl.pallas_call(
        _k_kf,
        grid=grid,
        in_specs=[
            pl.BlockSpec((BV, 64), row),
            pl.BlockSpec((8, 64), const),
            pl.BlockSpec((1, 64), const),
            pl.BlockSpec((1, 64), const),
            pl.BlockSpec((BV, 128), row),
            pl.BlockSpec((64, 64), const),
            pl.BlockSpec((128, 64), const),
            pl.BlockSpec((1, 64), const),
        ],
        out_specs=pl.BlockSpec((BV, 64), row),
        out_shape=jax.ShapeDtypeStruct((V, 64), F32),
    )(y2, st2, bn2_g.reshape(1, 64), bn2_b.reshape(1, 64), x_mod,
      K_w[:, :64].T, K_w[:, 64:].T, K_b.reshape(1, 64))

    q = pl.pallas_call(
        _k_q,
        grid=(1,),
        in_specs=[
            pl.BlockSpec((N, 128), const),
            pl.BlockSpec((128, 64), const),
            pl.BlockSpec((1, 64), const),
        ],
        out_specs=pl.BlockSpec((N, 64), const),
        out_shape=jax.ShapeDtypeStruct((N, 64), F32),
    )(x_main, Q_w.T, Q_b.reshape(1, 64))

    return kf, q


def _sc_body(kf_hbm, xm_hbm, q_hbm, csr_hbm, invs_hbm, out_hbm,
             csr_v, invs_v, q_v, kf_v, xm_v, out_v, m_v, s_v, mx_v):
    wid = lax.axis_index("s") * NC + lax.axis_index("c")
    n0 = wid * SPW
    pltpu.sync_copy(csr_hbm.at[pl.ds(n0, SPW + 16)], csr_v)
    pltpu.sync_copy(invs_hbm.at[pl.ds(n0, SPW + 16)], invs_v)
    pltpu.sync_copy(q_hbm.at[pl.ds(n0 * 64, SPW * 64)], q_v)

    zeros16 = jnp.zeros((16,), F32)
    ones16 = jnp.ones((16,), F32)
    NEG = F32(-3.4e38)

    def seg_body(jseg, _):
        csrvec = csr_v[pl.ds(jseg, 16)]
        lo = csrvec[0]
        hi = csrvec[1]
        iv16 = jnp.full((16,), invs_v[pl.ds(jseg, 16)][0])
        m_v[...] = jnp.full((16,), NEG)
        s_v[...] = zeros16
        for k in range(8):
            mx_v[pl.ds(16 * k, 16)] = jnp.full((16,), NEG)

        nch = (hi - lo + (CH - 1)) // CH

        def ch_body(c, _):
            cstart = lo + c * CH
            dstart = jnp.minimum(cstart, V - CH)
            doff = cstart - dstart
            jmax = jnp.minimum(CH, hi - cstart)
            pltpu.sync_copy(kf_hbm.at[pl.ds(dstart * 64, CH * 64)], kf_v)
            pltpu.sync_copy(xm_hbm.at[pl.ds(dstart * 128, CH * 128)], xm_v)

            def it_body(j, _):
                row = doff + j
                rb = row * 64
                qb = jseg * 64
                acc = kf_v[pl.ds(rb, 16)] * q_v[pl.ds(qb, 16)]
                acc = acc + kf_v[pl.ds(rb + 16, 16)] * q_v[pl.ds(qb + 16, 16)]
                acc = acc + kf_v[pl.ds(rb + 32, 16)] * q_v[pl.ds(qb + 32, 16)]
                acc = acc + kf_v[pl.ds(rb + 48, 16)] * q_v[pl.ds(qb + 48, 16)]
                xs = jnp.full((16,), jnp.sum(acc))
                mo = m_v[...]
                mn = jnp.maximum(mo, xs)
                rho = jnp.exp((mo - mn) * iv16)
                rho = jnp.where(mo < F32(-1e38), ones16, rho)
                e = jnp.exp((xs - mn) * iv16)
                s_v[...] = s_v[...] * rho + e
                m_v[...] = mn
                xb = row * 128
                for k in range(8):
                    sl = pl.ds(16 * k, 16)
                    mx_v[sl] = jnp.maximum(mx_v[sl] * rho,
                                           xm_v[pl.ds(xb + 16 * k, 16)] * e)
                return 0

            lax.fori_loop(0, jmax, it_body, 0)
            return 0

        lax.fori_loop(0, nch, ch_body, 0)

        t = jnp.maximum(m_v[...], zeros16)
        ex2 = jnp.exp(F32(-2.0) * t)
        gate = (ones16 - ex2) / (ones16 + ex2)
        factor = gate / (s_v[...] + F32(1e-12))
        ob = jseg * 128
        for k in range(8):
            out_v[pl.ds(ob + 16 * k, 16)] = mx_v[pl.ds(16 * k, 16)] * factor
        return 0

    lax.fori_loop(0, SPW, seg_body, 0)
    pltpu.sync_copy(out_v, out_hbm.at[pl.ds(n0 * 128, SPW * 128)])


def _sc_attend_pool(kf_flat, xm_flat, q_flat, csr_pad, invs_pad):
    mesh = plsc.VectorSubcoreMesh(core_axis_name="c", subcore_axis_name="s")
    cp = pltpu.CompilerParams()
    if "needs_layout_passes" in pltpu.CompilerParams.__dataclass_fields__:
        cp = dataclasses.replace(cp, needs_layout_passes=False)
    fn = pl.kernel(
        _sc_body,
        out_type=jax.ShapeDtypeStruct((NPAD * 128,), F32),
        mesh=mesh,
        compiler_params=cp,
        scratch_types=[
            pltpu.VMEM((SPW + 16,), jnp.int32),
            pltpu.VMEM((SPW + 16,), F32),
            pltpu.VMEM((SPW * 64,), F32),
            pltpu.VMEM((CH * 64,), F32),
            pltpu.VMEM((CH * 128,), F32),
            pltpu.VMEM((SPW * 128,), F32),
            pltpu.VMEM((16,), F32),
            pltpu.VMEM((16,), F32),
            pltpu.VMEM((128,), F32),
        ],
    )
    return fn(kf_flat, xm_flat, q_flat, csr_pad, invs_pad)


def kernel(x_main, x_mod, x_proj, csr_idx, Q_w, Q_b, p1_w, p1_b, bn1_g, bn1_b,
           p2_w, p2_b, bn2_g, bn2_b, K_w, K_b):
    csr = csr_idx.astype(jnp.int32)
    counts = csr[1:] - csr[:-1]
    seen = counts > 0
    invs = lax.rsqrt(jnp.maximum(counts, 1).astype(F32))

    kf, q = _dense_stage(x_main, x_mod, x_proj, Q_w, Q_b, p1_w, p1_b, bn1_g,
                         bn1_b, p2_w, p2_b, bn2_g, bn2_b, K_w, K_b)

    csr_pad = jnp.concatenate(
        [csr, jnp.full((CSR_PAD - (N + 1),), V, jnp.int32)])
    invs_pad = jnp.concatenate([invs, jnp.ones((NPAD + 16 - N,), F32)])
    q_pad = jnp.concatenate([q, jnp.zeros((NPAD - N, 64), F32)], axis=0)

    out_flat = _sc_attend_pool(kf.reshape(V * 64), x_mod.reshape(V * 128),
                               q_pad.reshape(NPAD * 64), csr_pad, invs_pad)
    x_pool = out_flat.reshape(NPAD, 128)[:N]
    return (x_pool, seen)


# speedup vs baseline: 11.0727x; 2.3500x over previous
import dataclasses
import functools

import jax
import jax.numpy as jnp
from jax import lax
from jax.experimental import pallas as pl
from jax.experimental.pallas import tpu as pltpu
from jax.experimental.pallas import tpu_sc as plsc

F32 = jnp.float32

N = 10000
V = 320000
BV = 2560

NC = 2
NS = 16
W = NC * NS
SPW = 320
NPAD = W * SPW
CSR_PAD = NPAD + 16
CH = 64
NCH = V // CH
KFB = 64 * CH
XMB = 128 * CH


def _k_y1(xp_ref, w_ref, b_ref, y1_ref, st_ref):
    i = pl.program_id(0)
    y = jnp.dot(xp_ref[...], w_ref[...], preferred_element_type=F32) + b_ref[...]
    y1_ref[...] = y

    @pl.when(i == 0)
    def _():
        st_ref[...] = jnp.zeros_like(st_ref)

    s = jnp.sum(y, axis=0, keepdims=True)
    sq = jnp.sum(y * y, axis=0, keepdims=True)
    pad = jnp.zeros((6, s.shape[1]), F32)
    st_ref[...] = st_ref[...] + jnp.concatenate([s, sq, pad], axis=0)


def _bn_affine(st_ref, g_ref, b_ref):
    inv_v = F32(1.0 / V)
    mean = st_ref[0:1, :] * inv_v
    var = st_ref[1:2, :] * inv_v - mean * mean
    a = g_ref[...] * lax.rsqrt(var + 1e-5)
    c = b_ref[...] - mean * a
    return a, c


def _k_y2(y1_ref, st1_ref, g1_ref, bb1_ref, w_ref, b_ref, y2_ref, st_ref):
    i = pl.program_id(0)
    a, c = _bn_affine(st1_ref, g1_ref, bb1_ref)
    h = jnp.maximum(y1_ref[...] * a + c, 0.0)
    y2 = jnp.dot(h, w_ref[...], preferred_element_type=F32) + b_ref[...]
    y2_ref[...] = y2

    @pl.when(i == 0)
    def _():
        st_ref[...] = jnp.zeros_like(st_ref)

    s = jnp.sum(y2, axis=0, keepdims=True)
    sq = jnp.sum(y2 * y2, axis=0, keepdims=True)
    pad = jnp.zeros((6, s.shape[1]), F32)
    st_ref[...] = st_ref[...] + jnp.concatenate([s, sq, pad], axis=0)


def _k_kf(y2_ref, st2_ref, g2_ref, bb2_ref, xm_ref, wh_ref, wm_ref, kb_ref,
          kf_ref):
    a, c = _bn_affine(st2_ref, g2_ref, bb2_ref)
    h2 = jnp.maximum(y2_ref[...] * a + c, 0.0)
    kf = jnp.dot(h2, wh_ref[...], preferred_element_type=F32)
    kf = kf + jnp.dot(xm_ref[...], wm_ref[...], preferred_element_type=F32)
    kf = kf + kb_ref[...]
    kf_ref[...] = kf.reshape(BV // CH, CH, 64).transpose(0, 2, 1)


def _k_q(xmain_ref, w_ref, b_ref, q_ref):
    q_ref[...] = (jnp.dot(xmain_ref[...], w_ref[...],
                          preferred_element_type=F32) + b_ref[...])


def _dense_stage(x_main, x_mod, x_proj, Q_w, Q_b, p1_w, p1_b, bn1_g, bn1_b,
                 p2_w, p2_b, bn2_g, bn2_b, K_w, K_b):
    grid = (V // BV,)
    row = lambda i: (i, 0)
    const = lambda i: (0, 0)

    y1, st1 = pl.pallas_call(
        _k_y1,
        grid=grid,
        in_specs=[
            pl.BlockSpec((BV, 128), row),
            pl.BlockSpec((128, 128), const),
            pl.BlockSpec((1, 128), const),
        ],
        out_specs=[
            pl.BlockSpec((BV, 128), row),
            pl.BlockSpec((8, 128), const),
        ],
        out_shape=[
            jax.ShapeDtypeStruct((V, 128), F32),
            jax.ShapeDtypeStruct((8, 128), F32),
        ],
    )(x_proj, p1_w.T, p1_b.reshape(1, 128))

    y2, st2 = pl.pallas_call(
        _k_y2,
        grid=grid,
        in_specs=[
            pl.BlockSpec((BV, 128), row),
            pl.BlockSpec((8, 128), const),
            pl.BlockSpec((1, 128), const),
            pl.BlockSpec((1, 128), const),
            pl.BlockSpec((128, 64), const),
            pl.BlockSpec((1, 64), const),
        ],
        out_specs=[
            pl.BlockSpec((BV, 64), row),
            pl.BlockSpec((8, 64), const),
        ],
        out_shape=[
            jax.ShapeDtypeStruct((V, 64), F32),
            jax.ShapeDtypeStruct((8, 64), F32),
        ],
    )(y1, st1, bn1_g.reshape(1, 128), bn1_b.reshape(1, 128), p2_w.T,
      p2_b.reshape(1, 64))

    kf = pl.pallas_call(
        _k_kf,
        grid=grid,
        in_specs=[
            pl.BlockSpec((BV, 64), row),
            pl.BlockSpec((8, 64), const),
            pl.BlockSpec((1, 64), const),
            pl.BlockSpec((1, 64), const),
            pl.BlockSpec((BV, 128), row),
            pl.BlockSpec((64, 64), const),
            pl.BlockSpec((128, 64), const),
            pl.BlockSpec((1, 64), const),
        ],
        out_specs=pl.BlockSpec((BV // CH, 64, CH), lambda i: (i, 0, 0)),
        out_shape=jax.ShapeDtypeStruct((NCH, 64, CH), F32),
    )(y2, st2, bn2_g.reshape(1, 64), bn2_b.reshape(1, 64), x_mod,
      K_w[:, :64].T, K_w[:, 64:].T, K_b.reshape(1, 64))

    q = pl.pallas_call(
        _k_q,
        grid=(1,),
        in_specs=[
            pl.BlockSpec((N, 128), const),
            pl.BlockSpec((128, 64), const),
            pl.BlockSpec((1, 64), const),
        ],
        out_specs=pl.BlockSpec((N, 64), const),
        out_shape=jax.ShapeDtypeStruct((N, 64), F32),
    )(x_main, Q_w.T, Q_b.reshape(1, 64))

    return kf, q


def _sc_body(kft_hbm, xm_hbm, q_hbm, csr_hbm, invs_hbm, out_hbm,
             csr_v, invs_v, q_v, qs_v, kf_b, xm_b, out_v, m_v, s_v, mx_v, e_v,
             sem_kf0, sem_kf1, sem_xm0, sem_xm1):
    wid = lax.axis_index("s") * NC + lax.axis_index("c")
    n0 = wid * SPW
    pltpu.sync_copy(csr_hbm.at[pl.ds(n0, SPW + 16)], csr_v)
    pltpu.sync_copy(invs_hbm.at[pl.ds(n0, SPW + 16)], invs_v)
    pltpu.sync_copy(q_hbm.at[pl.ds(n0 * 64, SPW * 64)], q_v)

    zeros16 = jnp.zeros((16,), F32)
    ones16 = jnp.ones((16,), F32)
    NEG = F32(-3.4e38)
    negv = jnp.full((16,), NEG)
    iota16 = jnp.arange(16, dtype=jnp.int32)

    def zo(i, carry):
        out_v[pl.ds(i * 16, 16)] = zeros16
        return carry

    lax.fori_loop(0, SPW * 8, zo, 0)

    m_v[...] = negv
    s_v[...] = zeros16
    for k in range(8):
        mx_v[pl.ds(16 * k, 16)] = negv

    lo_w = csr_v[pl.ds(0, 16)][0]
    hi_w = csr_v[pl.ds(SPW, 16)][0]
    c_lo = lo_w // CH
    c_hi = (hi_w - 1) // CH
    nch = jnp.maximum(c_hi - c_lo + 1, 0)

    def _issue(c, half):
        @pl.when(half == 0)
        def _():
            pltpu.async_copy(kft_hbm.at[pl.ds(c * KFB, KFB)],
                             kf_b.at[pl.ds(0, KFB)], sem_kf0)
            pltpu.async_copy(xm_hbm.at[pl.ds(c * XMB, XMB)],
                             xm_b.at[pl.ds(0, XMB)], sem_xm0)

        @pl.when(half == 1)
        def _():
            pltpu.async_copy(kft_hbm.at[pl.ds(c * KFB, KFB)],
                             kf_b.at[pl.ds(KFB, KFB)], sem_kf1)
            pltpu.async_copy(xm_hbm.at[pl.ds(c * XMB, XMB)],
                             xm_b.at[pl.ds(XMB, XMB)], sem_xm1)

    def _wait(half):
        @pl.when(half == 0)
        def _():
            pltpu.make_async_copy(kft_hbm.at[pl.ds(0, KFB)],
                                  kf_b.at[pl.ds(0, KFB)], sem_kf0).wait()
            pltpu.make_async_copy(xm_hbm.at[pl.ds(0, XMB)],
                                  xm_b.at[pl.ds(0, XMB)], sem_xm0).wait()

        @pl.when(half == 1)
        def _():
            pltpu.make_async_copy(kft_hbm.at[pl.ds(0, KFB)],
                                  kf_b.at[pl.ds(KFB, KFB)], sem_kf1).wait()
            pltpu.make_async_copy(xm_hbm.at[pl.ds(0, XMB)],
                                  xm_b.at[pl.ds(XMB, XMB)], sem_xm1).wait()

    @pl.when(nch > 0)
    def _():
        _issue(c_lo, 0)

    def chunk_body(c_rel, jseg):
        half = lax.rem(c_rel, 2)
        _wait(half)

        @pl.when(c_rel + 1 < nch)
        def _():
            _issue(c_lo + c_rel + 1, 1 - half)

        cbase = (c_lo + c_rel) * CH
        cend = cbase + CH
        kfo = half * KFB
        xmo = half * XMB

        def piece_cond(st):
            j, done = st
            return ((done == 0) & (j < SPW)
                    & (csr_v[pl.ds(j, 16)][0] < cend))

        def piece_body(st):
            j, done = st
            csr2 = csr_v[pl.ds(j, 16)]
            lo_s = csr2[0]
            hi_s = csr2[1]
            a = jnp.maximum(lo_s, cbase) - cbase
            b = jnp.minimum(hi_s, cend) - cbase
            iv16 = jnp.full((16,), invs_v[pl.ds(j, 16)][0])

            @pl.when(b > a)
            def _():
                for k4 in range(4):
                    q16 = q_v[pl.ds(j * 64 + 16 * k4, 16)]
                    for l in range(16):
                        qs_v[pl.ds((k4 * 16 + l) * 16, 16)] = (
                            jnp.full((16,), q16[l]))
                a16 = jnp.full((16,), a)
                b16 = jnp.full((16,), b)

                def grp_body(g, carry):
                    col = g * 16
                    col16 = iota16 + col
                    mask = (col16 >= a16) & (col16 < b16)
                    acc = [zeros16, zeros16, zeros16, zeros16]
                    for k in range(64):
                        acc[k % 4] = acc[k % 4] + (
                            kf_b[pl.ds(kfo + k * CH + col, 16)]
                            * qs_v[pl.ds(k * 16, 16)])
                    accs = (acc[0] + acc[1]) + (acc[2] + acc[3])
                    macc = jnp.where(mask, accs, negv)
                    gmx = jnp.full((16,), jnp.max(macc))
                    m16 = m_v[...]
                    mn = jnp.maximum(m16, gmx)
                    rho = jnp.exp((m16 - mn) * iv16)
                    rho = jnp.where(m16 < F32(-1e38), ones16, rho)
                    e16 = jnp.where(mask, jnp.exp((accs - mn) * iv16),
                                    zeros16)
                    s_v[...] = s_v[...] * rho + jnp.full((16,), jnp.sum(e16))
                    m_v[...] = mn
                    e_v[...] = e16
                    jlo = jnp.maximum(a, col)
                    jhi = jnp.minimum(b, col + 16)
                    ms = tuple(mx_v[pl.ds(16 * k, 16)] * rho
                               for k in range(8))

                    def it_body(jj, ms_):
                        es = plsc.load_gather(
                            e_v, [jnp.full((16,), jj - col, jnp.int32)])
                        xb = xmo + jj * 128
                        return tuple(
                            jnp.maximum(ms_[k],
                                        xm_b[pl.ds(xb + 16 * k, 16)] * es)
                            for k in range(8))

                    ms = lax.fori_loop(jlo, jhi, it_body, ms)
                    for k in range(8):
                        mx_v[pl.ds(16 * k, 16)] = ms[k]
                    return carry

                lax.fori_loop(a // 16, (b + 15) // 16, grp_body, 0)

            fin = hi_s <= cend

            @pl.when(fin)
            def _():
                t = jnp.maximum(m_v[...], zeros16)
                ex2 = jnp.exp(F32(-2.0) * t)
                gate = (ones16 - ex2) / (ones16 + ex2)
                factor = gate / (s_v[...] + F32(1e-12))
                ob = j * 128
                for k in range(8):
                    out_v[pl.ds(ob + 16 * k, 16)] = (
                        mx_v[pl.ds(16 * k, 16)] * factor)
                m_v[...] = negv
                s_v[...] = zeros16
                for k in range(8):
                    mx_v[pl.ds(16 * k, 16)] = negv

            j2 = jnp.where(fin, j + 1, j)
            done2 = jnp.where(fin, jnp.int32(0), jnp.int32(1))
            return (j2, done2)

        jseg2, _ = lax.while_loop(piece_cond, piece_body,
                                  (jseg, jnp.int32(0)))
        return jseg2

    lax.fori_loop(0, nch, chunk_body, jnp.int32(0))
    pltpu.sync_copy(out_v, out_hbm.at[pl.ds(n0 * 128, SPW * 128)])


def _sc_attend_pool(kf_flat, xm_flat, q_flat, csr_pad, invs_pad):
    mesh = plsc.VectorSubcoreMesh(core_axis_name="c", subcore_axis_name="s")
    cp = pltpu.CompilerParams()
    if "needs_layout_passes" in pltpu.CompilerParams.__dataclass_fields__:
        cp = dataclasses.replace(cp, needs_layout_passes=False)
    fn = pl.kernel(
        _sc_body,
        out_type=jax.ShapeDtypeStruct((NPAD * 128,), F32),
        mesh=mesh,
        compiler_params=cp,
        scratch_types=[
            pltpu.VMEM((SPW + 16,), jnp.int32),
            pltpu.VMEM((SPW + 16,), F32),
            pltpu.VMEM((SPW * 64,), F32),
            pltpu.VMEM((64 * 16,), F32),
            pltpu.VMEM((2 * KFB,), F32),
            pltpu.VMEM((2 * XMB,), F32),
            pltpu.VMEM((SPW * 128,), F32),
            pltpu.VMEM((16,), F32),
            pltpu.VMEM((16,), F32),
            pltpu.VMEM((128,), F32),
            pltpu.VMEM((16,), F32),
            pltpu.SemaphoreType.DMA,
            pltpu.SemaphoreType.DMA,
            pltpu.SemaphoreType.DMA,
            pltpu.SemaphoreType.DMA,
        ],
    )
    return fn(kf_flat, xm_flat, q_flat, csr_pad, invs_pad)


def kernel(x_main, x_mod, x_proj, csr_idx, Q_w, Q_b, p1_w, p1_b, bn1_g, bn1_b,
           p2_w, p2_b, bn2_g, bn2_b, K_w, K_b):
    csr = csr_idx.astype(jnp.int32)
    counts = csr[1:] - csr[:-1]
    seen = counts > 0
    invs = lax.rsqrt(jnp.maximum(counts, 1).astype(F32))

    kf, q = _dense_stage(x_main, x_mod, x_proj, Q_w, Q_b, p1_w, p1_b, bn1_g,
                         bn1_b, p2_w, p2_b, bn2_g, bn2_b, K_w, K_b)

    csr_pad = jnp.concatenate(
        [csr, jnp.full((CSR_PAD - (N + 1),), V, jnp.int32)])
    invs_pad = jnp.concatenate([invs, jnp.ones((NPAD + 16 - N,), F32)])
    q_pad = jnp.concatenate([q, jnp.zeros((NPAD - N, 64), F32)], axis=0)

    out_flat = _sc_attend_pool(kf.reshape(V * 64), x_mod.reshape(V * 128),
                               q_pad.reshape(NPAD * 64), csr_pad, invs_pad)
    x_pool = out_flat.reshape(NPAD, 128)[:N]
    return (x_pool, seen)
